# baseline (speedup 1.0000x reference)
import functools

import jax
import jax.numpy as jnp
import numpy as np
from jax.experimental import pallas as pl
from jax.experimental.pallas import tpu as pltpu

_N = 4096
_S = 64
_TOPK = 64
_NYQ = _N // 2


def _tables():
    a = np.arange(_S)
    m = np.outer(a, a).astype(np.float64)
    c64 = np.cos(2.0 * np.pi * m / _S)
    s64 = np.sin(2.0 * np.pi * m / _S)
    tc = np.cos(2.0 * np.pi * m / _N)
    ts = np.sin(2.0 * np.pi * m / _N)
    return (c64.astype(np.float32), s64.astype(np.float32),
            tc.astype(np.float32), ts.astype(np.float32))


_C64, _S64, _TC, _TS = _tables()


def _freq_body(x_ref, c_ref, s_ref, tc_ref, ts_ref, season_ref, trend_ref):
    r = x_ref.shape[0]
    x3 = x_ref[:]
    cm = c_ref[:]
    sm = s_ref[:]
    tc = tc_ref[:][None]
    ts = ts_ref[:][None]

    def mm(a3, b):
        a = a3.reshape(r * _S, _S)
        return jnp.dot(a, b, precision=jax.lax.Precision.HIGHEST,
                       preferred_element_type=jnp.float32).reshape(r, _S, _S)

    xt = jnp.swapaxes(x3, 1, 2)
    yr = mm(xt, cm)
    yi = -mm(xt, sm)
    zr = yr * tc + yi * ts
    zi = yi * tc - yr * ts
    zrt = jnp.swapaxes(zr, 1, 2)
    zit = jnp.swapaxes(zi, 1, 2)
    xr = mm(zrt, cm) + mm(zit, sm)
    xi = mm(zit, cm) - mm(zrt, sm)

    k1 = jax.lax.broadcasted_iota(jnp.int32, (1, _S, _S), 1)
    k2 = jax.lax.broadcasted_iota(jnp.int32, (1, _S, _S), 2)
    nat = k1 + _S * k2
    valid = nat <= _NYQ
    mag = xr * xr + xi * xi
    bits = jax.lax.bitcast_convert_type(mag, jnp.int32)
    bits = jnp.where(valid, bits, -1)

    def vstep(_, lh):
        lo, hi = lh
        d = hi - lo
        mid = lo + (d >> 1) + (d & 1)
        cnt = jnp.sum((bits >= mid).astype(jnp.int32), axis=(1, 2),
                      keepdims=True)
        p = cnt >= _TOPK
        return jnp.where(p, mid, lo), jnp.where(p, hi, mid - 1)

    lo0 = jnp.zeros((r, 1, 1), jnp.int32)
    hi0 = jnp.full((r, 1, 1), jnp.int32(2**31 - 1))
    v, _ = jax.lax.fori_loop(0, 31, vstep, (lo0, hi0))

    gt = bits > v
    eq = bits == v
    ngt = jnp.sum(gt.astype(jnp.int32), axis=(1, 2), keepdims=True)
    need = _TOPK - ngt

    def nstep(_, lh):
        lo, hi = lh
        d = hi - lo
        mid = lo + (d >> 1) + (d & 1)
        cnt = jnp.sum((eq & (nat <= mid)).astype(jnp.int32), axis=(1, 2),
                      keepdims=True)
        p = cnt <= need
        return jnp.where(p, mid, lo), jnp.where(p, hi, mid - 1)

    jlo0 = jnp.full((r, 1, 1), -1, jnp.int32)
    jhi0 = jnp.full((r, 1, 1), _N - 1, jnp.int32)
    jsel, _ = jax.lax.fori_loop(0, 13, nstep, (jlo0, jhi0))
    keep = gt | (eq & (nat <= jsel))

    w = jnp.where((nat == 0) | (nat == _NYQ), 1.0, 2.0) * (1.0 / _N)
    w = jnp.where(valid, w, 0.0)
    wk = jnp.where(keep, w, 0.0)
    gr = xr * wk
    gi = xi * wk
    ar = mm(gr, cm) - mm(gi, sm)
    ai = mm(gr, sm) + mm(gi, cm)
    br = ar * tc - ai * ts
    bi = ar * ts + ai * tc
    brt = jnp.swapaxes(br, 1, 2)
    bit_ = jnp.swapaxes(bi, 1, 2)
    tr = mm(brt, cm) - mm(bit_, sm)
    trend = jnp.swapaxes(tr, 1, 2)
    trend_ref[:] = trend
    season_ref[:] = x3 - trend


@functools.partial(jax.jit, static_argnames=("block_rows", "interpret"))
def _freq2d(x2, block_rows=32, interpret=False):
    rows = x2.shape[0]
    nb = rows // block_rows
    x3 = x2.reshape(rows, _S, _S)
    full = pl.BlockSpec((block_rows, _S, _S), lambda i: (i, 0, 0))
    const = lambda: pl.BlockSpec((_S, _S), lambda i: (0, 0))
    season, trend = pl.pallas_call(
        _freq_body,
        grid=(nb,),
        in_specs=[full, const(), const(), const(), const()],
        out_specs=[full, full],
        out_shape=[jax.ShapeDtypeStruct((rows, _S, _S), jnp.float32),
                   jax.ShapeDtypeStruct((rows, _S, _S), jnp.float32)],
        compiler_params=pltpu.CompilerParams(
            dimension_semantics=("parallel",)),
        interpret=interpret,
    )(x3, _C64, _S64, _TC, _TS)
    return season.reshape(rows, _N), trend.reshape(rows, _N)


def kernel(x):
    shp = x.shape
    x2 = x.reshape(-1, _N)
    season, trend = _freq2d(x2)
    return season.reshape(shp), trend.reshape(shp)


# speedup vs baseline: 1.2453x; 1.2453x over previous
import functools

import jax
import jax.numpy as jnp
import numpy as np
from jax.experimental import pallas as pl
from jax.experimental.pallas import tpu as pltpu

_N = 4096
_S = 64
_TOPK = 64
_NYQ = _N // 2


def _tables():
    a = np.arange(_S)
    m = np.outer(a, a).astype(np.float64)
    c = np.cos(2.0 * np.pi * m / _S)
    s = np.sin(2.0 * np.pi * m / _S)
    tc = np.cos(2.0 * np.pi * m / _N)
    ts = np.sin(2.0 * np.pi * m / _N)
    f32 = lambda z: np.ascontiguousarray(z, np.float32)
    cs_a = f32(np.concatenate([c, -s], axis=1))
    w_b = f32(np.block([[c, -s], [s, c]]))
    w_c = f32(np.block([[c, s], [-s, c]]))
    w_d = f32(np.concatenate([c, -s], axis=0))
    t1 = f32(np.concatenate([tc, tc], axis=1))
    t2 = f32(np.concatenate([ts, -ts], axis=1))
    return cs_a, w_b, w_c, w_d, t1, t2


_CSA, _WB, _WC, _WD, _T1, _T2 = _tables()


def _halfswap(a):
    return jnp.concatenate([a[:, :, _S:], a[:, :, :_S]], axis=2)


def _repack(a):
    return jnp.concatenate([a[:, :_S, :], a[:, _S:, :]], axis=2)


def _freq_body(x_ref, csa_ref, wb_ref, wc_ref, wd_ref, t1_ref, t2_ref,
               season_ref, trend_ref):
    r = x_ref.shape[0]
    x3 = x_ref[:]
    csa = csa_ref[:]
    wb = wb_ref[:]
    wc = wc_ref[:]
    wd = wd_ref[:]
    t1 = t1_ref[:][None]
    t2 = t2_ref[:][None]

    def mm(a3, b):
        m = a3.shape[0] * a3.shape[1]
        out = jnp.dot(a3.reshape(m, a3.shape[2]), b,
                      precision=jax.lax.Precision.HIGHEST,
                      preferred_element_type=jnp.float32)
        return out.reshape(r, _S, b.shape[1])

    xt = jnp.swapaxes(x3, 1, 2)
    y = mm(xt, csa)
    z = y * t1 + _halfswap(y) * t2
    zc = _repack(jnp.swapaxes(z, 1, 2))
    xp = mm(zc, wb)
    xr = xp[:, :, :_S]
    xi = xp[:, :, _S:]

    k1 = jax.lax.broadcasted_iota(jnp.int32, (1, _S, _S), 1)
    k2 = jax.lax.broadcasted_iota(jnp.int32, (1, _S, _S), 2)
    nat = k1 + _S * k2
    valid = nat <= _NYQ
    mag = xr * xr + xi * xi
    bits = jax.lax.bitcast_convert_type(mag, jnp.int32)
    bits = jnp.where(valid, bits, -1)

    def vstep(_, lh):
        lo, hi = lh
        d = hi - lo
        mid = lo + (d >> 1) + (d & 1)
        cnt = jnp.sum((bits >= mid).astype(jnp.int32), axis=(1, 2),
                      keepdims=True)
        p = cnt >= _TOPK
        return jnp.where(p, mid, lo), jnp.where(p, hi, mid - 1)

    lo0 = jnp.zeros((r, 1, 1), jnp.int32)
    hi0 = jnp.full((r, 1, 1), jnp.int32(2**31 - 1))
    v, _ = jax.lax.fori_loop(0, 31, vstep, (lo0, hi0))

    gt = bits > v
    eq = bits == v
    ngt = jnp.sum(gt.astype(jnp.int32), axis=(1, 2), keepdims=True)
    need = _TOPK - ngt

    def nstep(_, lh):
        lo, hi = lh
        d = hi - lo
        mid = lo + (d >> 1) + (d & 1)
        cnt = jnp.sum((eq & (nat <= mid)).astype(jnp.int32), axis=(1, 2),
                      keepdims=True)
        p = cnt <= need
        return jnp.where(p, mid, lo), jnp.where(p, hi, mid - 1)

    jlo0 = jnp.full((r, 1, 1), -1, jnp.int32)
    jhi0 = jnp.full((r, 1, 1), _N - 1, jnp.int32)
    jsel, _ = jax.lax.fori_loop(0, 13, nstep, (jlo0, jhi0))
    keep = gt | (eq & (nat <= jsel))

    w = jnp.where((nat == 0) | (nat == _NYQ), 1.0, 2.0) * (1.0 / _N)
    w = jnp.where(valid, w, 0.0)
    wk = jnp.where(keep, w, 0.0)
    wk2 = jnp.concatenate([wk, wk], axis=2)
    g = xp * wk2
    a = mm(g, wc)
    b = a * t1 - _halfswap(a) * t2
    bc = _repack(jnp.swapaxes(b, 1, 2))
    tr = mm(bc, wd)
    trend = jnp.swapaxes(tr, 1, 2)
    trend_ref[:] = trend
    season_ref[:] = x3 - trend


@functools.partial(jax.jit, static_argnames=("block_rows", "interpret"))
def _freq2d(x2, block_rows=64, interpret=False):
    rows = x2.shape[0]
    nb = rows // block_rows
    x3 = x2.reshape(rows, _S, _S)
    full = pl.BlockSpec((block_rows, _S, _S), lambda i: (i, 0, 0))
    const = lambda shape: pl.BlockSpec(shape, lambda i: (0, 0))
    season, trend = pl.pallas_call(
        _freq_body,
        grid=(nb,),
        in_specs=[full,
                  const((_S, 2 * _S)), const((2 * _S, 2 * _S)),
                  const((2 * _S, 2 * _S)), const((2 * _S, _S)),
                  const((_S, 2 * _S)), const((_S, 2 * _S))],
        out_specs=[full, full],
        out_shape=[jax.ShapeDtypeStruct((rows, _S, _S), jnp.float32),
                   jax.ShapeDtypeStruct((rows, _S, _S), jnp.float32)],
        compiler_params=pltpu.CompilerParams(
            dimension_semantics=("parallel",)),
        interpret=interpret,
    )(x3, _CSA, _WB, _WC, _WD, _T1, _T2)
    return season.reshape(rows, _N), trend.reshape(rows, _N)


def kernel(x):
    shp = x.shape
    x2 = x.reshape(-1, _N)
    season, trend = _freq2d(x2)
    return season.reshape(shp), trend.reshape(shp)
